# Initial kernel scaffold; baseline (speedup 1.0000x reference)
import jax
import jax.numpy as jnp
from jax.experimental import pallas as pl


def kernel(x, centroids, weights):
    raise NotImplementedError("write your pallas kernel here")


# speedup vs baseline: 2.7226x; 2.7226x over previous
import functools

import jax
import jax.numpy as jnp
from jax.experimental import pallas as pl


def _asmk_kernel(x_ref, c_ref, w_ref, out_ref):
    x = x_ref[0]
    c = c_ref[...]
    n = x.shape[0]

    x2 = jnp.sum(x * x, axis=1, keepdims=True)
    c2 = jnp.sum(c * c, axis=1)[None, :]
    xc = jax.lax.dot_general(
        x, c, (((1,), (1,)), ((), ())),
        preferred_element_type=jnp.float32)
    d2 = jnp.maximum(x2 + c2 - 2.0 * xc, 0.0)
    dist = jnp.sqrt(d2)

    nearest = jnp.argmin(dist, axis=1)
    min_d = jnp.min(dist, axis=1)

    mean = jnp.mean(min_d)
    std = jnp.sqrt(jnp.sum((min_d - mean) ** 2) / (n - 1))
    thr = mean + std
    mask = (min_d < thr).astype(jnp.float32)

    kiota = jax.lax.broadcasted_iota(jnp.int32, dist.shape, 1)
    onehot = (nearest[:, None] == kiota)
    hist = jnp.sum(jnp.where(onehot, mask[:, None], 0.0), axis=0)

    asmk = w_ref[0] * hist
    norm = jnp.sqrt(jnp.sum(asmk * asmk))
    out_ref[0, 0, :] = asmk / jnp.maximum(norm, 1e-12)


@functools.partial(jax.jit, static_argnames=())
def kernel(x, centroids, weights):
    B, N, D = x.shape
    K = centroids.shape[0]
    w2d = weights.reshape(1, K)
    return pl.pallas_call(
        _asmk_kernel,
        grid=(B,),
        in_specs=[
            pl.BlockSpec((1, N, D), lambda b: (b, 0, 0)),
            pl.BlockSpec((K, D), lambda b: (0, 0)),
            pl.BlockSpec((1, K), lambda b: (0, 0)),
        ],
        out_specs=pl.BlockSpec((1, 1, K), lambda b: (b, 0, 0)),
        out_shape=jax.ShapeDtypeStruct((B, 1, K), x.dtype),
    )(x, centroids, w2d).reshape(B, K)


# speedup vs baseline: 3.2557x; 1.1958x over previous
import functools

import jax
import jax.numpy as jnp
from jax.experimental import pallas as pl
from jax.experimental.pallas import tpu as pltpu


def _asmk_kernel(x_ref, c_ref, w_ref, out_ref):
    x = x_ref[0]
    c = c_ref[...]
    n = x.shape[0]

    x2 = jnp.sum(x * x, axis=1, keepdims=True)
    c2 = jnp.sum(c * c, axis=1)[None, :]
    xcn = jax.lax.dot_general(
        x * -2.0, c, (((1,), (1,)), ((), ())),
        preferred_element_type=jnp.float32)
    d2 = jnp.maximum((x2 + c2) + xcn, 0.0)

    nearest = jnp.argmin(d2, axis=1)
    min_d = jnp.sqrt(jnp.min(d2, axis=1))

    mean = jnp.mean(min_d)
    std = jnp.sqrt(jnp.sum((min_d - mean) ** 2) / (n - 1))
    thr = mean + std
    mask = (min_d < thr).astype(jnp.float32)

    kiota = jax.lax.broadcasted_iota(jnp.int32, d2.shape, 1)
    onehot = (nearest[:, None] == kiota)
    hist = jnp.sum(jnp.where(onehot, mask[:, None], 0.0), axis=0)

    asmk = w_ref[0] * hist
    norm = jnp.sqrt(jnp.sum(asmk * asmk))
    out_ref[0, 0, :] = asmk / jnp.maximum(norm, 1e-12)


@functools.partial(jax.jit, static_argnames=())
def kernel(x, centroids, weights):
    B, N, D = x.shape
    K = centroids.shape[0]
    w2d = weights.reshape(1, K)
    return pl.pallas_call(
        _asmk_kernel,
        grid=(B,),
        in_specs=[
            pl.BlockSpec((1, N, D), lambda b: (b, 0, 0)),
            pl.BlockSpec((K, D), lambda b: (0, 0)),
            pl.BlockSpec((1, K), lambda b: (0, 0)),
        ],
        out_specs=pl.BlockSpec((1, 1, K), lambda b: (b, 0, 0)),
        out_shape=jax.ShapeDtypeStruct((B, 1, K), x.dtype),
        compiler_params=pltpu.CompilerParams(
            dimension_semantics=("parallel",)),
    )(x, centroids, w2d).reshape(B, K)


# speedup vs baseline: 4.0297x; 1.2377x over previous
import functools

import jax
import jax.numpy as jnp
from jax.experimental import pallas as pl
from jax.experimental.pallas import tpu as pltpu

_BB = 2


def _asmk_kernel(x_ref, c_ref, w_ref, out_ref):
    bb, n, d = x_ref.shape
    k = c_ref.shape[0]
    x = x_ref[...].reshape(bb * n, d)
    c = c_ref[...]

    x2 = jnp.sum(x * x, axis=1, keepdims=True)
    c2 = jnp.sum(c * c, axis=1)[None, :]
    xcn = jax.lax.dot_general(
        x * -2.0, c, (((1,), (1,)), ((), ())),
        preferred_element_type=jnp.float32)
    d2 = (x2 + c2) + xcn

    nearest = jnp.argmin(d2, axis=1)
    min_d = jnp.sqrt(jnp.maximum(jnp.min(d2, axis=1), 0.0))

    md = min_d.reshape(bb, n)
    mean = jnp.mean(md, axis=1, keepdims=True)
    std = jnp.sqrt(jnp.sum((md - mean) ** 2, axis=1, keepdims=True) / (n - 1))
    thr = mean + std
    mask = (md < thr).astype(jnp.float32).reshape(bb * n)

    kiota = jax.lax.broadcasted_iota(jnp.int32, (bb * n, k), 1)
    onehot = (nearest[:, None] == kiota)
    contrib = jnp.where(onehot, mask[:, None], 0.0)
    hist = jnp.sum(contrib.reshape(bb, n, k), axis=1)

    asmk = w_ref[...] * hist
    norm = jnp.sqrt(jnp.sum(asmk * asmk, axis=1, keepdims=True))
    out_ref[...] = (asmk / jnp.maximum(norm, 1e-12)).reshape(bb, 1, k)


@functools.partial(jax.jit, static_argnames=())
def kernel(x, centroids, weights):
    B, N, D = x.shape
    K = centroids.shape[0]
    w2d = weights.reshape(1, K)
    return pl.pallas_call(
        _asmk_kernel,
        grid=(B // _BB,),
        in_specs=[
            pl.BlockSpec((_BB, N, D), lambda b: (b, 0, 0)),
            pl.BlockSpec((K, D), lambda b: (0, 0)),
            pl.BlockSpec((1, K), lambda b: (0, 0)),
        ],
        out_specs=pl.BlockSpec((_BB, 1, K), lambda b: (b, 0, 0)),
        out_shape=jax.ShapeDtypeStruct((B, 1, K), x.dtype),
        compiler_params=pltpu.CompilerParams(
            dimension_semantics=("parallel",)),
    )(x, centroids, w2d).reshape(B, K)
